# Initial kernel scaffold; baseline (speedup 1.0000x reference)
import jax
import jax.numpy as jnp
from jax.experimental import pallas as pl


def kernel(semantic_indices, semantic_history):
    raise NotImplementedError("write your pallas kernel here")


# speedup vs baseline: 3.1309x; 3.1309x over previous
import functools

import jax
import jax.numpy as jnp
from jax import lax
from jax.experimental import pallas as pl
from jax.experimental.pallas import tpu as pltpu
import jax.experimental.pallas.tpu_sc as plsc

N_TOKENS = 16384
N_SYMBOLS = 8192
WEIGHT = 0.1

NUM_CORES = 2
NUM_SUBCORES = 16
NUM_WORKERS = NUM_CORES * NUM_SUBCORES
TOK_PER_WORKER = N_TOKENS // NUM_WORKERS
IDX_ROWS = TOK_PER_WORKER // 128
BINS_PER_SUBCORE = N_SYMBOLS // NUM_SUBCORES


def _sc_hist_body(idx_hbm, out_hbm, idx_v, ones_v, zeros_v, hist_sh, sem):
    del sem
    cid = lax.axis_index("c")
    sid = lax.axis_index("s")
    wid = cid * NUM_SUBCORES + sid

    for j in range(IDX_ROWS):
        for k in range(128 // 16):
            ones_v[j, pl.ds(k * 16, 16)] = jnp.ones((16,), jnp.float32)
    for k in range(BINS_PER_SUBCORE // 16):
        zeros_v[pl.ds(k * 16, 16)] = jnp.zeros((16,), jnp.float32)

    pltpu.sync_copy(zeros_v, hist_sh.at[pl.ds(sid * BINS_PER_SUBCORE, BINS_PER_SUBCORE)])
    pltpu.sync_copy(idx_hbm.at[wid], idx_v)
    plsc.subcore_barrier()

    for j in range(IDX_ROWS):
        pltpu.sync_copy(ones_v.at[j], hist_sh.at[idx_v.at[j]], add=True)
    plsc.subcore_barrier()

    pltpu.sync_copy(
        hist_sh.at[pl.ds(sid * BINS_PER_SUBCORE, BINS_PER_SUBCORE)],
        out_hbm.at[cid, pl.ds(sid * BINS_PER_SUBCORE, BINS_PER_SUBCORE)],
    )


@functools.partial(
    pl.kernel,
    out_type=jax.ShapeDtypeStruct((NUM_CORES, N_SYMBOLS), jnp.float32),
    mesh=plsc.VectorSubcoreMesh(core_axis_name="c", subcore_axis_name="s"),
    scratch_types=[
        pltpu.VMEM((IDX_ROWS, 128), jnp.int32),
        pltpu.VMEM((IDX_ROWS, 128), jnp.float32),
        pltpu.VMEM((BINS_PER_SUBCORE,), jnp.float32),
        pltpu.VMEM_SHARED((N_SYMBOLS,), jnp.float32),
        pltpu.SemaphoreType.DMA,
    ],
)
def _sc_hist(idx_hbm, out_hbm, idx_v, ones_v, zeros_v, hist_sh, sem):
    _sc_hist_body(idx_hbm, out_hbm, idx_v, ones_v, zeros_v, hist_sh, sem)


def _tc_loss_body(counts_ref, hist_ref, out_ref):
    c = counts_ref[...]
    rows = N_SYMBOLS // 128
    counts = c[:rows, :] + c[rows:, :]
    new_history = hist_ref[...] * 0.95 + counts * (0.05 / N_TOKENS)
    total = jnp.sum(new_history)
    probs = new_history / (total + 1e-9)
    loss = WEIGHT * jnp.sum(probs * jnp.log(probs + 1e-9))
    out_ref[...] = jnp.full((1, 1), loss, jnp.float32)


def kernel(semantic_indices, semantic_history):
    idx = semantic_indices.reshape(NUM_WORKERS, IDX_ROWS, 128).astype(jnp.int32)
    partials = _sc_hist(idx)
    counts2 = partials.reshape(NUM_CORES * (N_SYMBOLS // 128), 128)
    hist2 = semantic_history.reshape(N_SYMBOLS // 128, 128)
    loss = pl.pallas_call(
        _tc_loss_body,
        out_shape=jax.ShapeDtypeStruct((1, 1), jnp.float32),
    )(counts2, hist2)
    return loss[0, 0]


# speedup vs baseline: 3.4143x; 1.0905x over previous
import functools

import jax
import jax.numpy as jnp
from jax import lax
from jax.experimental import pallas as pl
from jax.experimental.pallas import tpu as pltpu
import jax.experimental.pallas.tpu_sc as plsc

N_TOKENS = 16384
N_SYMBOLS = 8192
WEIGHT = 0.1

NUM_SUBCORES = 16
TOK_PER_SC = N_TOKENS // NUM_SUBCORES
IDX_ROWS = TOK_PER_SC // 128
BINS_PER_SC = N_SYMBOLS // NUM_SUBCORES
VECS_PER_SC = BINS_PER_SC // 16

_LN2 = 0.6931471805599453
_SQRT2 = 1.4142135623730951


def _vlog(x):
    bits = lax.bitcast_convert_type(x, jnp.int32)
    e = lax.shift_right_logical(bits, 23) - 127
    m = lax.bitcast_convert_type(
        (bits & jnp.int32(0x007FFFFF)) | jnp.int32(0x3F800000), jnp.float32)
    big = m >= _SQRT2
    e = jnp.where(big, e + 1, e)
    m = jnp.where(big, m * 0.5, m)
    t = (m - 1.0) / (m + 1.0)
    t2 = t * t
    ln_m = t * (2.0 + t2 * (2.0 / 3.0 + t2 * (2.0 / 5.0 + t2 * (2.0 / 7.0))))
    return ln_m + e.astype(jnp.float32) * _LN2


def _hsum_splat(x):
    acc = jnp.full((16,), x[0], jnp.float32)
    for l in range(1, 16):
        acc = acc + jnp.full((16,), x[l], jnp.float32)
    return acc


def _sc_body(idx_hbm, histy_hbm, out_hbm,
             idx_v, ones_v, cnt_v, histy_v, red_v, out_v,
             hist_sh, sums_sh, ents_sh, sem):
    del sem
    sid = lax.axis_index("s")

    scope = jax.named_scope
    with scope("phase_stage"):
        for j in range(IDX_ROWS):
            for k in range(128 // 16):
                ones_v[j, pl.ds(k * 16, 16)] = jnp.ones((16,), jnp.float32)
        for k in range(VECS_PER_SC):
            cnt_v[pl.ds(k * 16, 16)] = jnp.zeros((16,), jnp.float32)

        pltpu.sync_copy(cnt_v, hist_sh.at[pl.ds(sid * BINS_PER_SC, BINS_PER_SC)])
        pltpu.sync_copy(idx_hbm.at[sid], idx_v)
        pltpu.sync_copy(histy_hbm.at[pl.ds(sid * BINS_PER_SC, BINS_PER_SC)], histy_v)
        plsc.subcore_barrier()

    with scope("phase_scatter"):
        for j in range(IDX_ROWS):
            pltpu.sync_copy(ones_v.at[j], hist_sh.at[idx_v.at[j]], add=True)
        plsc.subcore_barrier()

    with scope("phase_ema"):
        pltpu.sync_copy(hist_sh.at[pl.ds(sid * BINS_PER_SC, BINS_PER_SC)], cnt_v)

        acc = jnp.zeros((16,), jnp.float32)
        for k in range(VECS_PER_SC):
            h = histy_v[pl.ds(k * 16, 16)] * 0.95 + cnt_v[pl.ds(k * 16, 16)] * (0.05 / N_TOKENS)
            cnt_v[pl.ds(k * 16, 16)] = h
            acc = acc + h
        out_v[...] = acc
        pltpu.sync_copy(out_v, sums_sh.at[pl.ds(sid * 16, 16)])
        plsc.subcore_barrier()

    with scope("phase_entropy"):
        pltpu.sync_copy(sums_sh, red_v)
        tot = jnp.zeros((16,), jnp.float32)
        for r in range(NUM_SUBCORES):
            tot = tot + red_v[pl.ds(r * 16, 16)]
        inv = 1.0 / (_hsum_splat(tot) + 1e-9)

        ent = jnp.zeros((16,), jnp.float32)
        for k in range(VECS_PER_SC):
            p = cnt_v[pl.ds(k * 16, 16)] * inv
            ent = ent + p * _vlog(p + 1e-9)
        out_v[...] = ent
        pltpu.sync_copy(out_v, ents_sh.at[pl.ds(sid * 16, 16)])
        plsc.subcore_barrier()

    with scope("phase_final"):
        _final(sid, ents_sh, red_v, out_v, out_hbm)


def _final(sid, ents_sh, red_v, out_v, out_hbm):
    @pl.when(sid == 0)
    def _():
        pltpu.sync_copy(ents_sh, red_v)
        tot2 = jnp.zeros((16,), jnp.float32)
        for r in range(NUM_SUBCORES):
            tot2 = tot2 + red_v[pl.ds(r * 16, 16)]
        out_v[...] = _hsum_splat(tot2) * WEIGHT
        pltpu.sync_copy(out_v, out_hbm)


@functools.partial(
    pl.kernel,
    out_type=jax.ShapeDtypeStruct((16,), jnp.float32),
    mesh=plsc.VectorSubcoreMesh(core_axis_name="c", subcore_axis_name="s",
                                num_cores=1),
    scratch_types=[
        pltpu.VMEM((IDX_ROWS, 128), jnp.int32),
        pltpu.VMEM((IDX_ROWS, 128), jnp.float32),
        pltpu.VMEM((BINS_PER_SC,), jnp.float32),
        pltpu.VMEM((BINS_PER_SC,), jnp.float32),
        pltpu.VMEM((NUM_SUBCORES * 16,), jnp.float32),
        pltpu.VMEM((16,), jnp.float32),
        pltpu.VMEM_SHARED((N_SYMBOLS,), jnp.float32),
        pltpu.VMEM_SHARED((NUM_SUBCORES * 16,), jnp.float32),
        pltpu.VMEM_SHARED((NUM_SUBCORES * 16,), jnp.float32),
        pltpu.SemaphoreType.DMA,
    ],
)
def _sc_loss(idx_hbm, histy_hbm, out_hbm, *rest):
    _sc_body(idx_hbm, histy_hbm, out_hbm, *rest)


def kernel(semantic_indices, semantic_history):
    idx = semantic_indices.reshape(NUM_SUBCORES, IDX_ROWS, 128).astype(jnp.int32)
    out = _sc_loss(idx, semantic_history)
    return out[0]


# speedup vs baseline: 3.4931x; 1.0231x over previous
import functools

import jax
import jax.numpy as jnp
from jax import lax
from jax.experimental import pallas as pl
from jax.experimental.pallas import tpu as pltpu
import jax.experimental.pallas.tpu_sc as plsc

N_TOKENS = 16384
N_SYMBOLS = 8192
WEIGHT = 0.1

NUM_SUBCORES = 16
TOK_PER_SC = N_TOKENS // NUM_SUBCORES
IDX_ROWS = TOK_PER_SC // 128
BINS_PER_SC = N_SYMBOLS // NUM_SUBCORES
VECS_PER_SC = BINS_PER_SC // 16

_LN2 = 0.6931471805599453
_SQRT2 = 1.4142135623730951


def _vlog(x):
    bits = lax.bitcast_convert_type(x, jnp.int32)
    e = lax.shift_right_logical(bits, 23) - 127
    m = lax.bitcast_convert_type(
        (bits & jnp.int32(0x007FFFFF)) | jnp.int32(0x3F800000), jnp.float32)
    big = m >= _SQRT2
    e = jnp.where(big, e + 1, e)
    m = jnp.where(big, m * 0.5, m)
    t = (m - 1.0) / (m + 1.0)
    t2 = t * t
    ln_m = t * (2.0 + t2 * (2.0 / 3.0 + t2 * (2.0 / 5.0 + t2 * (2.0 / 7.0))))
    return ln_m + e.astype(jnp.float32) * _LN2


def _hsum_splat(x):
    acc = jnp.full((16,), x[0], jnp.float32)
    for l in range(1, 16):
        acc = acc + jnp.full((16,), x[l], jnp.float32)
    return acc


def _sc_body(idx_hbm, histy_hbm, out_hbm,
             idx_v, ones_v, cnt_v, histy_v, red_v, out_v,
             hist_sh, sums_sh, ents_sh, sem):
    sid = lax.axis_index("s")

    scope = jax.named_scope
    with scope("phase_stage"):
        for j in range(IDX_ROWS):
            for k in range(128 // 16):
                ones_v[j, pl.ds(k * 16, 16)] = jnp.ones((16,), jnp.float32)
        for k in range(VECS_PER_SC):
            cnt_v[pl.ds(k * 16, 16)] = jnp.zeros((16,), jnp.float32)

        pltpu.sync_copy(cnt_v, hist_sh.at[pl.ds(sid * BINS_PER_SC, BINS_PER_SC)])
        pltpu.sync_copy(idx_hbm.at[sid], idx_v)
        pltpu.sync_copy(histy_hbm.at[pl.ds(sid * BINS_PER_SC, BINS_PER_SC)], histy_v)
        plsc.subcore_barrier()

    with scope("phase_scatter"):
        scatters = [
            pltpu.async_copy(ones_v.at[j], hist_sh.at[idx_v.at[j]], sem,
                             add=True)
            for j in range(IDX_ROWS)
        ]
        for c in scatters:
            c.wait()
        plsc.subcore_barrier()

    with scope("phase_ema"):
        pltpu.sync_copy(hist_sh.at[pl.ds(sid * BINS_PER_SC, BINS_PER_SC)], cnt_v)

        acc = jnp.zeros((16,), jnp.float32)
        for k in range(VECS_PER_SC):
            h = histy_v[pl.ds(k * 16, 16)] * 0.95 + cnt_v[pl.ds(k * 16, 16)] * (0.05 / N_TOKENS)
            cnt_v[pl.ds(k * 16, 16)] = h
            acc = acc + h
        out_v[...] = acc
        pltpu.sync_copy(out_v, sums_sh.at[pl.ds(sid * 16, 16)])
        plsc.subcore_barrier()

    with scope("phase_entropy"):
        pltpu.sync_copy(sums_sh, red_v)
        tot = jnp.zeros((16,), jnp.float32)
        for r in range(NUM_SUBCORES):
            tot = tot + red_v[pl.ds(r * 16, 16)]
        inv = 1.0 / (_hsum_splat(tot) + 1e-9)

        ent = jnp.zeros((16,), jnp.float32)
        for k in range(VECS_PER_SC):
            p = cnt_v[pl.ds(k * 16, 16)] * inv
            ent = ent + p * _vlog(p + 1e-9)
        out_v[...] = ent
        pltpu.sync_copy(out_v, ents_sh.at[pl.ds(sid * 16, 16)])
        plsc.subcore_barrier()

    with scope("phase_final"):
        _final(sid, ents_sh, red_v, out_v, out_hbm)


def _final(sid, ents_sh, red_v, out_v, out_hbm):
    @pl.when(sid == 0)
    def _():
        pltpu.sync_copy(ents_sh, red_v)
        tot2 = jnp.zeros((16,), jnp.float32)
        for r in range(NUM_SUBCORES):
            tot2 = tot2 + red_v[pl.ds(r * 16, 16)]
        out_v[...] = _hsum_splat(tot2) * WEIGHT
        pltpu.sync_copy(out_v, out_hbm)


@functools.partial(
    pl.kernel,
    out_type=jax.ShapeDtypeStruct((16,), jnp.float32),
    mesh=plsc.VectorSubcoreMesh(core_axis_name="c", subcore_axis_name="s",
                                num_cores=1),
    scratch_types=[
        pltpu.VMEM((IDX_ROWS, 128), jnp.int32),
        pltpu.VMEM((IDX_ROWS, 128), jnp.float32),
        pltpu.VMEM((BINS_PER_SC,), jnp.float32),
        pltpu.VMEM((BINS_PER_SC,), jnp.float32),
        pltpu.VMEM((NUM_SUBCORES * 16,), jnp.float32),
        pltpu.VMEM((16,), jnp.float32),
        pltpu.VMEM_SHARED((N_SYMBOLS,), jnp.float32),
        pltpu.VMEM_SHARED((NUM_SUBCORES * 16,), jnp.float32),
        pltpu.VMEM_SHARED((NUM_SUBCORES * 16,), jnp.float32),
        pltpu.SemaphoreType.DMA,
    ],
)
def _sc_loss(idx_hbm, histy_hbm, out_hbm, *rest):
    _sc_body(idx_hbm, histy_hbm, out_hbm, *rest)


def kernel(semantic_indices, semantic_history):
    idx = semantic_indices.reshape(NUM_SUBCORES, IDX_ROWS, 128).astype(jnp.int32)
    out = _sc_loss(idx, semantic_history)
    return out[0]


# speedup vs baseline: 3.5560x; 1.0180x over previous
import functools

import jax
import jax.numpy as jnp
from jax import lax
from jax.experimental import pallas as pl
from jax.experimental.pallas import tpu as pltpu
import jax.experimental.pallas.tpu_sc as plsc

N_TOKENS = 16384
N_SYMBOLS = 8192
WEIGHT = 0.1

NUM_SUBCORES = 16
TOK_PER_SC = N_TOKENS // NUM_SUBCORES
IDX_ROWS = TOK_PER_SC // 128
BINS_PER_SC = N_SYMBOLS // NUM_SUBCORES
VECS_PER_SC = BINS_PER_SC // 16

_LN2 = 0.6931471805599453
_SQRT2 = 1.4142135623730951
_TINY = 2e-38


def _vlog(x):
    bits = lax.bitcast_convert_type(x, jnp.int32)
    e = lax.shift_right_logical(bits, 23) - 127
    m = lax.bitcast_convert_type(
        (bits & jnp.int32(0x007FFFFF)) | jnp.int32(0x3F800000), jnp.float32)
    big = m >= _SQRT2
    e = jnp.where(big, e + 1, e)
    m = jnp.where(big, m * 0.5, m)
    t = (m - 1.0) / (m + 1.0)
    t2 = t * t
    ln_m = t * (2.0 + t2 * (2.0 / 3.0 + t2 * (2.0 / 5.0 + t2 * (2.0 / 7.0))))
    return ln_m + e.astype(jnp.float32) * _LN2


def _hsum_splat(x):
    acc = jnp.full((16,), x[0], jnp.float32)
    for l in range(1, 16):
        acc = acc + jnp.full((16,), x[l], jnp.float32)
    return acc


def _sc_body(idx_hbm, histy_hbm, out_hbm,
             idx_v, ones_v, cnt_v, histy_v, red_v, out_v,
             hist_sh, sums_sh, ents_sh, sem):
    sid = lax.axis_index("s")

    for j in range(IDX_ROWS):
        for k in range(128 // 16):
            ones_v[j, pl.ds(k * 16, 16)] = jnp.ones((16,), jnp.float32)
    for k in range(VECS_PER_SC):
        cnt_v[pl.ds(k * 16, 16)] = jnp.zeros((16,), jnp.float32)

    pltpu.sync_copy(cnt_v, hist_sh.at[pl.ds(sid * BINS_PER_SC, BINS_PER_SC)])
    pltpu.sync_copy(idx_hbm.at[sid], idx_v)
    pltpu.sync_copy(histy_hbm.at[pl.ds(sid * BINS_PER_SC, BINS_PER_SC)], histy_v)
    plsc.subcore_barrier()

    scatters = [
        pltpu.async_copy(ones_v.at[j], hist_sh.at[idx_v.at[j]], sem, add=True)
        for j in range(IDX_ROWS)
    ]
    for c in scatters:
        c.wait()
    plsc.subcore_barrier()

    pltpu.sync_copy(hist_sh.at[pl.ds(sid * BINS_PER_SC, BINS_PER_SC)], cnt_v)

    acc = jnp.zeros((16,), jnp.float32)
    for k in range(VECS_PER_SC):
        h = (histy_v[pl.ds(k * 16, 16)] * 0.95
             + cnt_v[pl.ds(k * 16, 16)] * (0.05 / N_TOKENS))
        cnt_v[pl.ds(k * 16, 16)] = h
        acc = acc + h
    out_v[...] = acc
    c_sum = pltpu.async_copy(out_v, sums_sh.at[pl.ds(sid * 16, 16)], sem)
    e1 = jnp.zeros((16,), jnp.float32)
    for k in range(VECS_PER_SC):
        h = cnt_v[pl.ds(k * 16, 16)]
        e1 = e1 + h * _vlog(h + _TINY)
    c_sum.wait()
    plsc.subcore_barrier()

    pltpu.sync_copy(sums_sh, red_v)
    tot = jnp.zeros((16,), jnp.float32)
    for r in range(NUM_SUBCORES):
        tot = tot + red_v[pl.ds(r * 16, 16)]
    inv = 1.0 / (_hsum_splat(tot) + 1e-9)

    out_v[...] = inv * (e1 + _vlog(inv) * acc)
    pltpu.sync_copy(out_v, ents_sh.at[pl.ds(sid * 16, 16)])
    plsc.subcore_barrier()

    @pl.when(sid == 0)
    def _():
        pltpu.sync_copy(ents_sh, red_v)
        tot2 = jnp.zeros((16,), jnp.float32)
        for r in range(NUM_SUBCORES):
            tot2 = tot2 + red_v[pl.ds(r * 16, 16)]
        out_v[...] = _hsum_splat(tot2) * WEIGHT
        pltpu.sync_copy(out_v, out_hbm)


@functools.partial(
    pl.kernel,
    out_type=jax.ShapeDtypeStruct((16,), jnp.float32),
    mesh=plsc.VectorSubcoreMesh(core_axis_name="c", subcore_axis_name="s",
                                num_cores=1),
    scratch_types=[
        pltpu.VMEM((IDX_ROWS, 128), jnp.int32),
        pltpu.VMEM((IDX_ROWS, 128), jnp.float32),
        pltpu.VMEM((BINS_PER_SC,), jnp.float32),
        pltpu.VMEM((BINS_PER_SC,), jnp.float32),
        pltpu.VMEM((NUM_SUBCORES * 16,), jnp.float32),
        pltpu.VMEM((16,), jnp.float32),
        pltpu.VMEM_SHARED((N_SYMBOLS,), jnp.float32),
        pltpu.VMEM_SHARED((NUM_SUBCORES * 16,), jnp.float32),
        pltpu.VMEM_SHARED((NUM_SUBCORES * 16,), jnp.float32),
        pltpu.SemaphoreType.DMA,
    ],
)
def _sc_loss(idx_hbm, histy_hbm, out_hbm, *rest):
    _sc_body(idx_hbm, histy_hbm, out_hbm, *rest)


def kernel(semantic_indices, semantic_history):
    idx = semantic_indices.reshape(NUM_SUBCORES, IDX_ROWS, 128).astype(jnp.int32)
    out = _sc_loss(idx, semantic_history)
    return out[0]


# speedup vs baseline: 3.5638x; 1.0022x over previous
import functools

import jax
import jax.numpy as jnp
from jax import lax
from jax.experimental import pallas as pl
from jax.experimental.pallas import tpu as pltpu
import jax.experimental.pallas.tpu_sc as plsc

N_TOKENS = 16384
N_SYMBOLS = 8192
WEIGHT = 0.1

NUM_SUBCORES = 16
TOK_PER_SC = N_TOKENS // NUM_SUBCORES
IDX_ROWS = TOK_PER_SC // 128
BINS_PER_SC = N_SYMBOLS // NUM_SUBCORES
VECS_PER_SC = BINS_PER_SC // 16

_LN2 = 0.6931471805599453
_SQRT2 = 1.4142135623730951
_TINY = 2e-38


def _vlog(x):
    bits = lax.bitcast_convert_type(x, jnp.int32)
    e = lax.shift_right_logical(bits, 23) - 127
    m = lax.bitcast_convert_type(
        (bits & jnp.int32(0x007FFFFF)) | jnp.int32(0x3F800000), jnp.float32)
    big = m >= _SQRT2
    e = jnp.where(big, e + 1, e)
    m = jnp.where(big, m * 0.5, m)
    t = (m - 1.0) / (m + 1.0)
    t2 = t * t
    ln_m = t * (2.0 + t2 * (2.0 / 3.0 + t2 * (2.0 / 5.0 + t2 * (2.0 / 7.0))))
    return ln_m + e.astype(jnp.float32) * _LN2


def _hsum_splat(x):
    acc = jnp.full((16,), x[0], jnp.float32)
    for l in range(1, 16):
        acc = acc + jnp.full((16,), x[l], jnp.float32)
    return acc


def _sc_body(idx_hbm, histy_hbm, out_hbm,
             idx_v, ones_v, cnt_v, histy_v, iota_v, out_v,
             hist_sh, acc_sh, sem):
    sid = lax.axis_index("s")

    for j in range(IDX_ROWS):
        for k in range(128 // 16):
            ones_v[j, pl.ds(k * 16, 16)] = jnp.ones((16,), jnp.float32)
    for k in range(VECS_PER_SC):
        cnt_v[pl.ds(k * 16, 16)] = jnp.zeros((16,), jnp.float32)
    iota_v[pl.ds(0, 16)] = lax.iota(jnp.int32, 16)
    iota_v[pl.ds(16, 16)] = lax.iota(jnp.int32, 16) + 16

    pltpu.sync_copy(cnt_v, hist_sh.at[pl.ds(sid * BINS_PER_SC, BINS_PER_SC)])

    @pl.when(sid == 0)
    def _():
        pltpu.sync_copy(cnt_v.at[pl.ds(0, 32)], acc_sh)

    pltpu.sync_copy(idx_hbm.at[sid], idx_v)
    pltpu.sync_copy(histy_hbm.at[pl.ds(sid * BINS_PER_SC, BINS_PER_SC)], histy_v)
    plsc.subcore_barrier()

    scatters = [
        pltpu.async_copy(ones_v.at[j], hist_sh.at[idx_v.at[j]], sem, add=True)
        for j in range(IDX_ROWS)
    ]
    for c in scatters:
        c.wait()
    plsc.subcore_barrier()

    pltpu.sync_copy(hist_sh.at[pl.ds(sid * BINS_PER_SC, BINS_PER_SC)], cnt_v)

    acc = jnp.zeros((16,), jnp.float32)
    for k in range(VECS_PER_SC):
        h = (histy_v[pl.ds(k * 16, 16)] * 0.95
             + cnt_v[pl.ds(k * 16, 16)] * (0.05 / N_TOKENS))
        cnt_v[pl.ds(k * 16, 16)] = h
        acc = acc + h
    out_v[pl.ds(0, 16)] = acc
    e1 = jnp.zeros((16,), jnp.float32)
    for k in range(VECS_PER_SC):
        h = cnt_v[pl.ds(k * 16, 16)]
        e1 = e1 + h * _vlog(h + _TINY)
    out_v[pl.ds(16, 16)] = e1
    pltpu.sync_copy(out_v, acc_sh.at[iota_v], add=True)
    plsc.subcore_barrier()

    @pl.when(sid == 0)
    def _():
        pltpu.sync_copy(acc_sh, cnt_v.at[pl.ds(0, 32)])
        e2_tot = _hsum_splat(cnt_v[pl.ds(0, 16)])
        e1_tot = _hsum_splat(cnt_v[pl.ds(16, 16)])
        inv = 1.0 / (e2_tot + 1e-9)
        out_v[pl.ds(0, 16)] = WEIGHT * inv * (e1_tot + _vlog(inv) * e2_tot)
        pltpu.sync_copy(out_v.at[pl.ds(0, 16)], out_hbm)


@functools.partial(
    pl.kernel,
    out_type=jax.ShapeDtypeStruct((16,), jnp.float32),
    mesh=plsc.VectorSubcoreMesh(core_axis_name="c", subcore_axis_name="s",
                                num_cores=1),
    scratch_types=[
        pltpu.VMEM((IDX_ROWS, 128), jnp.int32),
        pltpu.VMEM((IDX_ROWS, 128), jnp.float32),
        pltpu.VMEM((BINS_PER_SC,), jnp.float32),
        pltpu.VMEM((BINS_PER_SC,), jnp.float32),
        pltpu.VMEM((32,), jnp.int32),
        pltpu.VMEM((32,), jnp.float32),
        pltpu.VMEM_SHARED((N_SYMBOLS,), jnp.float32),
        pltpu.VMEM_SHARED((32,), jnp.float32),
        pltpu.SemaphoreType.DMA,
    ],
)
def _sc_loss(idx_hbm, histy_hbm, out_hbm, *rest):
    _sc_body(idx_hbm, histy_hbm, out_hbm, *rest)


def kernel(semantic_indices, semantic_history):
    idx = semantic_indices.reshape(NUM_SUBCORES, IDX_ROWS, 128).astype(jnp.int32)
    out = _sc_loss(idx, semantic_history)
    return out[0]


# speedup vs baseline: 3.6719x; 1.0303x over previous
import functools

import jax
import jax.numpy as jnp
from jax import lax
from jax.experimental import pallas as pl
from jax.experimental.pallas import tpu as pltpu
import jax.experimental.pallas.tpu_sc as plsc

N_TOKENS = 16384
N_SYMBOLS = 8192
WEIGHT = 0.1

NUM_SUBCORES = 16
TOK_PER_SC = N_TOKENS // NUM_SUBCORES
IDX_ROWS = TOK_PER_SC // 128
BINS_PER_SC = N_SYMBOLS // NUM_SUBCORES
VECS_PER_SC = BINS_PER_SC // 16

_LN2 = 0.6931471805599453
_SQRT2 = 1.4142135623730951
_TINY = 2e-38


def _vlog(x):
    bits = lax.bitcast_convert_type(x, jnp.int32)
    e = lax.shift_right_logical(bits, 23) - 127
    m = lax.bitcast_convert_type(
        (bits & jnp.int32(0x007FFFFF)) | jnp.int32(0x3F800000), jnp.float32)
    big = m >= _SQRT2
    e = jnp.where(big, e + 1, e)
    m = jnp.where(big, m * 0.5, m)
    t = (m - 1.0) / (m + 1.0)
    t2 = t * t
    ln_m = t * (2.0 + t2 * (2.0 / 3.0 + t2 * (2.0 / 5.0 + t2 * (2.0 / 7.0))))
    return ln_m + e.astype(jnp.float32) * _LN2


def _hsum_splat(x):
    acc = jnp.full((16,), x[0], jnp.float32)
    for l in range(1, 16):
        acc = acc + jnp.full((16,), x[l], jnp.float32)
    return acc


def _sc_body(idx_hbm, histy_hbm, out_hbm,
             idx_v, ones_v, cnt_v, histy_v, iota_v, out_v,
             hist_sh, acc_sh, sem):
    sid = lax.axis_index("s")

    for j in range(IDX_ROWS):
        for k in range(128 // 16):
            ones_v[j, pl.ds(k * 16, 16)] = jnp.ones((16,), jnp.float32)
    for k in range(VECS_PER_SC):
        cnt_v[pl.ds(k * 16, 16)] = jnp.zeros((16,), jnp.float32)
    iota_v[pl.ds(0, 16)] = lax.iota(jnp.int32, 16)
    iota_v[pl.ds(16, 16)] = lax.iota(jnp.int32, 16) + 16

    c_idx = pltpu.async_copy(idx_hbm.at[sid], idx_v, sem)
    c_hty = pltpu.async_copy(
        histy_hbm.at[pl.ds(sid * BINS_PER_SC, BINS_PER_SC)], histy_v, sem)
    pltpu.sync_copy(cnt_v, hist_sh.at[pl.ds(sid * BINS_PER_SC, BINS_PER_SC)])

    @pl.when(sid == 0)
    def _():
        pltpu.sync_copy(cnt_v.at[pl.ds(0, 32)], acc_sh)

    c_idx.wait()
    c_hty.wait()
    plsc.subcore_barrier()

    scatters = [
        pltpu.async_copy(ones_v.at[j], hist_sh.at[idx_v.at[j]], sem, add=True)
        for j in range(IDX_ROWS)
    ]
    for c in scatters:
        c.wait()
    plsc.subcore_barrier()

    pltpu.sync_copy(hist_sh.at[pl.ds(sid * BINS_PER_SC, BINS_PER_SC)], cnt_v)

    acc = jnp.zeros((16,), jnp.float32)
    for k in range(VECS_PER_SC):
        h = (histy_v[pl.ds(k * 16, 16)] * 0.95
             + cnt_v[pl.ds(k * 16, 16)] * (0.05 / N_TOKENS))
        cnt_v[pl.ds(k * 16, 16)] = h
        acc = acc + h
    out_v[pl.ds(0, 16)] = acc
    e1 = jnp.zeros((16,), jnp.float32)
    for k in range(VECS_PER_SC):
        h = cnt_v[pl.ds(k * 16, 16)]
        e1 = e1 + h * _vlog(h + _TINY)
    out_v[pl.ds(16, 16)] = e1
    pltpu.sync_copy(out_v, acc_sh.at[iota_v], add=True)
    plsc.subcore_barrier()

    @pl.when(sid == 0)
    def _():
        pltpu.sync_copy(acc_sh, cnt_v.at[pl.ds(0, 32)])
        e2_tot = _hsum_splat(cnt_v[pl.ds(0, 16)])
        e1_tot = _hsum_splat(cnt_v[pl.ds(16, 16)])
        inv = 1.0 / (e2_tot + 1e-9)
        out_v[pl.ds(0, 16)] = WEIGHT * inv * (e1_tot + _vlog(inv) * e2_tot)
        pltpu.sync_copy(out_v.at[pl.ds(0, 16)], out_hbm)


@functools.partial(
    pl.kernel,
    out_type=jax.ShapeDtypeStruct((16,), jnp.float32),
    mesh=plsc.VectorSubcoreMesh(core_axis_name="c", subcore_axis_name="s",
                                num_cores=1),
    scratch_types=[
        pltpu.VMEM((IDX_ROWS, 128), jnp.int32),
        pltpu.VMEM((IDX_ROWS, 128), jnp.float32),
        pltpu.VMEM((BINS_PER_SC,), jnp.float32),
        pltpu.VMEM((BINS_PER_SC,), jnp.float32),
        pltpu.VMEM((32,), jnp.int32),
        pltpu.VMEM((32,), jnp.float32),
        pltpu.VMEM_SHARED((N_SYMBOLS,), jnp.float32),
        pltpu.VMEM_SHARED((32,), jnp.float32),
        pltpu.SemaphoreType.DMA,
    ],
)
def _sc_loss(idx_hbm, histy_hbm, out_hbm, *rest):
    _sc_body(idx_hbm, histy_hbm, out_hbm, *rest)


def kernel(semantic_indices, semantic_history):
    idx = semantic_indices.reshape(NUM_SUBCORES, IDX_ROWS, 128).astype(jnp.int32)
    out = _sc_loss(idx, semantic_history)
    return out[0]
